# Initial kernel scaffold; baseline (speedup 1.0000x reference)
import jax
import jax.numpy as jnp
from jax.experimental import pallas as pl


def kernel(hidden_states, gate_up_weight_packed, gate_up_scales, down_weight_packed, down_scales, expert_ids, expert_probs):
    raise NotImplementedError("write your pallas kernel here")


# speedup vs baseline: 1.7742x; 1.7742x over previous
import functools

import jax
import jax.numpy as jnp
from jax.experimental import pallas as pl
from jax.experimental.pallas import tpu as pltpu

E = 16
K = 2048
I = 1024
GS = 32
T = 64
TOPK = 2

NT = 512


def _decode_fp4(nib):
    n = nib.astype(jnp.int32)
    m3 = n & 7
    ex = m3 >> 1
    man = (m3 & 1).astype(jnp.float32)
    pow2 = jnp.left_shift(1, jnp.maximum(ex - 1, 0)).astype(jnp.float32)
    base = jnp.where(ex > 0, 1.0, 0.0) + 0.5 * man
    mag = base * pow2
    return jnp.where(n >= 8, -mag, mag)


def _dequant_tile(packed, scales):
    kp, n = packed.shape
    pu = packed.astype(jnp.uint32)
    shifts = (jnp.arange(8, dtype=jnp.uint32) * 4)[None, :, None]
    nib = (pu[:, None, :] >> shifts) & jnp.uint32(0xF)
    vals = _decode_fp4(nib)
    srep = jnp.repeat(scales, 4, axis=0)
    w3 = vals * srep[:, None, :]
    return w3.reshape(kp * 8, n)


def _moe_kernel(x_ref, ids_ref, probs_ref, gup_ref, gups_ref, dwn_ref,
                dwns_ref, out_ref):
    e = pl.program_id(0)

    @pl.when(e == 0)
    def _init():
        out_ref[...] = jnp.zeros_like(out_ref)

    x = x_ref[...]

    h = []
    for nt in range(2 * I // NT):
        w = _dequant_tile(gup_ref[0, :, nt * NT:(nt + 1) * NT],
                          gups_ref[0, :, nt * NT:(nt + 1) * NT])
        h.append(jax.lax.dot_general(
            x, w, (((1,), (0,)), ((), ())),
            preferred_element_type=jnp.float32))

    n_half = I // NT
    ids = ids_ref[...]
    probs = probs_ref[...]
    w_tok = jnp.sum(jnp.where(ids == e, probs, 0.0), axis=1)

    acts = []
    for j in range(n_half):
        g = h[j]
        u = h[j + n_half]
        a = (g * jax.nn.sigmoid(g)) * u
        acts.append(a * w_tok[:, None])

    ip_half = (I // 8) // n_half
    is_half = (I // GS) // n_half
    for kt in range(K // NT):
        y = jnp.zeros((T, NT), dtype=jnp.float32)
        for j in range(n_half):
            wd = _dequant_tile(
                dwn_ref[0, j * ip_half:(j + 1) * ip_half,
                        kt * NT:(kt + 1) * NT],
                dwns_ref[0, j * is_half:(j + 1) * is_half,
                         kt * NT:(kt + 1) * NT])
            y = y + jax.lax.dot_general(
                acts[j], wd, (((1,), (0,)), ((), ())),
                preferred_element_type=jnp.float32)
        out_ref[:, kt * NT:(kt + 1) * NT] += y


@jax.jit
def kernel(hidden_states, gate_up_weight_packed, gate_up_scales,
           down_weight_packed, down_scales, expert_ids, expert_probs):
    grid = (E,)
    out = pl.pallas_call(
        _moe_kernel,
        grid=grid,
        in_specs=[
            pl.BlockSpec((T, K), lambda e: (0, 0)),
            pl.BlockSpec((T, TOPK), lambda e: (0, 0)),
            pl.BlockSpec((T, TOPK), lambda e: (0, 0)),
            pl.BlockSpec((1, K // 8, 2 * I), lambda e: (e, 0, 0)),
            pl.BlockSpec((1, K // GS, 2 * I), lambda e: (e, 0, 0)),
            pl.BlockSpec((1, I // 8, K), lambda e: (e, 0, 0)),
            pl.BlockSpec((1, I // GS, K), lambda e: (e, 0, 0)),
        ],
        out_specs=pl.BlockSpec((T, K), lambda e: (0, 0)),
        out_shape=jax.ShapeDtypeStruct((T, K), jnp.float32),
        compiler_params=pltpu.CompilerParams(
            dimension_semantics=("arbitrary",),
        ),
    )(hidden_states, expert_ids, expert_probs,
      gate_up_weight_packed, gate_up_scales,
      down_weight_packed, down_scales)
    return out
